# Initial kernel scaffold; baseline (speedup 1.0000x reference)
import jax
import jax.numpy as jnp
from jax.experimental import pallas as pl


def kernel(x, edge_index, edge_attr, W_ne, b_ne, W_ee, b_ee, Wc0, bc0, Wc1, bc1, Wc2, bc2, Wd1, bd1, Wd2, bd2, Wi1, bi1, Wi2, bi2):
    raise NotImplementedError("write your pallas kernel here")


# speedup vs baseline: 9.9255x; 9.9255x over previous
import functools

import jax
import jax.numpy as jnp
from jax import lax
from jax.experimental import pallas as pl
from jax.experimental.pallas import tpu as pltpu
from jax.experimental.pallas import tpu_sc as plsc

_N = 100000
_H = 64
_NTILES = 16
_NPAD = 100352
_STRIPE = _NPAD // _NTILES
_E = 1600000

_BLK = 128
_SBB = 4
_NSB = 200
_BPT = _SBB * _NSB
_EPAD = _NTILES * _BPT * _BLK
_NBLK = _EPAD // _BLK
_NSB2 = _NSB // 2
_DROWS = 392


def _fill(buf, rows, value):
    vec = jnp.full((16,), value, jnp.float32)

    def body(i, _):
        buf[i, :] = vec
        return 0

    lax.fori_loop(0, rows, body, 0)


def _zero_accum(accum, zbuf, s):
    def body(k, _):
        pltpu.sync_copy(zbuf, accum.at[pl.ds(s * _STRIPE + k * _DROWS, _DROWS)])
        return 0

    lax.fori_loop(0, _STRIPE // _DROWS, body, 0)


def _dump_accum(accum, dbuf, out, out_row0, s):
    def body(k, _):
        r0 = s * _STRIPE + k * _DROWS
        pltpu.sync_copy(accum.at[pl.ds(r0, _DROWS)], dbuf)
        pltpu.sync_copy(dbuf, out.at[pl.ds(out_row0 + r0, _DROWS)])
        return 0

    lax.fori_loop(0, _STRIPE // _DROWS, body, 0)


def _sc_degree_body(dstp2, out, didx, obuf, dbuf, accum, sem_i, sem_s):
    c = lax.axis_index("c")
    s = lax.axis_index("s")
    _fill(obuf, _BLK, 1.0)
    _fill(dbuf, _DROWS, 0.0)
    _zero_accum(accum, dbuf, s)
    plsc.subcore_barrier()

    base = (c * _NTILES + s) * (_NSB2 * _SBB)

    def fire_idx(i, slot):
        pltpu.async_copy(dstp2.at[pl.ds(base + i * _SBB, _SBB)],
                         didx.at[slot], sem_i)

    def fire_scatters(slot):
        for b in range(_SBB):
            pltpu.async_copy(obuf, accum.at[didx.at[slot].at[b]], sem_s,
                             add=True)

    def drain_idx(slot):
        pltpu.make_async_copy(dstp2.at[pl.ds(0, _SBB)], didx.at[slot],
                              sem_i).wait()

    def drain_scatters(slot):
        for b in range(_SBB):
            pltpu.make_async_copy(obuf, accum.at[didx.at[slot].at[b]],
                                  sem_s).wait()

    fire_idx(0, 0)
    drain_idx(0)
    fire_idx(1, 1)
    fire_scatters(0)

    def body(i, _):
        slot = lax.rem(i, 2)
        prev = 1 - slot
        drain_idx(slot)

        @pl.when(i < _NSB2 - 1)
        def _():
            fire_idx(i + 1, prev)

        drain_scatters(prev)
        fire_scatters(slot)
        return 0

    lax.fori_loop(1, _NSB2, body, 0)
    drain_scatters((_NSB2 - 1) % 2)
    plsc.subcore_barrier()
    _dump_accum(accum, dbuf, out, c * _NPAD, s)


def _sc_edge_body(g4, srcp4, dstp2, out, sidx, didx, rows, dbuf, accum,
                  sem_i, sem_g, sem_s):
    c = lax.axis_index("c")
    s = lax.axis_index("s")
    base = s * _BPT

    for chunk in range(2):
        cidx = c * 2 + chunk
        _fill(dbuf, _DROWS, 0.0)
        _zero_accum(accum, dbuf, s)
        plsc.subcore_barrier()

        def fire_idx(i, slot):
            blk0 = base + i * _SBB
            pltpu.async_copy(srcp4.at[cidx].at[pl.ds(blk0, _SBB)],
                             sidx.at[slot], sem_i)
            pltpu.async_copy(dstp2.at[pl.ds(blk0, _SBB)],
                             didx.at[slot], sem_i)

        def drain_idx(slot):
            pltpu.make_async_copy(dstp2.at[pl.ds(0, _SBB)], sidx.at[slot],
                                  sem_i).wait()
            pltpu.make_async_copy(dstp2.at[pl.ds(0, _SBB)], didx.at[slot],
                                  sem_i).wait()

        def fire_gathers(slot):
            for b in range(_SBB):
                pltpu.async_copy(g4.at[sidx.at[slot].at[b]],
                                 rows.at[slot].at[b], sem_g)

        def drain_gathers(slot):
            for b in range(_SBB):
                pltpu.make_async_copy(g4.at[sidx.at[slot].at[b]],
                                      rows.at[slot].at[b], sem_g).wait()

        def fire_scatters(slot):
            for b in range(_SBB):
                pltpu.async_copy(rows.at[slot].at[b],
                                 accum.at[didx.at[slot].at[b]], sem_s,
                                 add=True)

        def drain_scatters(slot):
            for b in range(_SBB):
                pltpu.make_async_copy(rows.at[slot].at[b],
                                      accum.at[didx.at[slot].at[b]],
                                      sem_s).wait()

        fire_idx(0, 0)
        drain_idx(0)
        fire_idx(1, 1)
        fire_gathers(0)
        drain_gathers(0)
        fire_scatters(0)

        def body(i, _):
            slot = lax.rem(i, 2)
            prev = 1 - slot
            drain_idx(slot)

            @pl.when(i < _NSB - 1)
            def _():
                fire_idx(i + 1, prev)

            fire_gathers(slot)
            drain_scatters(prev)
            drain_gathers(slot)
            fire_scatters(slot)
            return 0

        lax.fori_loop(1, _NSB, body, 0)
        drain_scatters((_NSB - 1) % 2)
        plsc.subcore_barrier()
        _dump_accum(accum, dbuf, out, cidx * _NPAD, s)
        plsc.subcore_barrier()


@functools.cache
def _sc_kernels():
    mesh = plsc.VectorSubcoreMesh(
        core_axis_name="c", subcore_axis_name="s", num_cores=2, num_subcores=16
    )
    params = pltpu.CompilerParams(use_tc_tiling_on_sc=False)
    sc_degree = pl.kernel(
        _sc_degree_body,
        out_type=jax.ShapeDtypeStruct((2 * _NPAD, 16), jnp.float32),
        mesh=mesh,
        scratch_types=[
            pltpu.VMEM((2, _SBB, _BLK), jnp.int32),
            pltpu.VMEM((_BLK, 16), jnp.float32),
            pltpu.VMEM((_DROWS, 16), jnp.float32),
            pltpu.VMEM_SHARED((_NPAD, 16), jnp.float32),
            pltpu.SemaphoreType.DMA,
            pltpu.SemaphoreType.DMA,
        ],
        compiler_params=params,
    )
    sc_edge_pass = pl.kernel(
        _sc_edge_body,
        out_type=jax.ShapeDtypeStruct((4 * _NPAD, 16), jnp.float32),
        mesh=mesh,
        scratch_types=[
            pltpu.VMEM((2, _SBB, _BLK), jnp.int32),
            pltpu.VMEM((2, _SBB, _BLK), jnp.int32),
            pltpu.VMEM((2, _SBB, _BLK, 16), jnp.float32),
            pltpu.VMEM((_DROWS, 16), jnp.float32),
            pltpu.VMEM_SHARED((_NPAD, 16), jnp.float32),
            pltpu.SemaphoreType.DMA,
            pltpu.SemaphoreType.DMA,
            pltpu.SemaphoreType.DMA,
        ],
        compiler_params=params,
    )
    return sc_degree, sc_edge_pass


_RB = 2000
_GRID = _N // _RB


def _tc1_body(x_ref, wne_ref, bne_ref, wc_ref, dis_ref, g_ref):
    h = jnp.maximum(x_ref[...] @ wne_ref[...] + bne_ref[...], 0.0)
    g_ref[...] = (h @ wc_ref[...]) * dis_ref[...]


def _tc2_body(s_ref, g_ref, dis_ref, b_ref, wc_ref, o_ref):
    dis = dis_ref[...]
    h = jnp.maximum(dis * (s_ref[...] + g_ref[...]) + b_ref[...], 0.0)
    o_ref[...] = (h @ wc_ref[...]) * dis


def _tc3_body(s_ref, g_ref, dis_ref, b_ref, wd1_ref, bd1_ref, wd2_ref,
              bd2_ref, wi1_ref, bi1_ref, wi2_ref, bi2_ref, d_ref, i_ref):
    dis = dis_ref[...]
    h = jnp.maximum(dis * (s_ref[...] + g_ref[...]) + b_ref[...], 0.0)
    t = jnp.maximum(h @ wd1_ref[...] + bd1_ref[...], 0.0)
    d_ref[...] = t @ wd2_ref[...] + bd2_ref[...]
    u = jnp.maximum(h @ wi1_ref[...] + bi1_ref[...], 0.0)
    i_ref[...] = u @ wi2_ref[...] + bi2_ref[...]


def _row_spec(cols):
    return pl.BlockSpec((_RB, cols), lambda i: (i, 0))


def _full_spec(r, c):
    return pl.BlockSpec((r, c), lambda i: (0, 0))


def _tc_encode(x, wne, bne, wc0, dis):
    return pl.pallas_call(
        _tc1_body,
        grid=(_GRID,),
        in_specs=[
            _row_spec(5), _full_spec(5, _H), _full_spec(1, _H),
            _full_spec(_H, _H), _row_spec(1),
        ],
        out_specs=_row_spec(_H),
        out_shape=jax.ShapeDtypeStruct((_N, _H), jnp.float32),
    )(x, wne, bne, wc0, dis)


def _tc_layer(sacc, g, dis, b, wc):
    return pl.pallas_call(
        _tc2_body,
        grid=(_GRID,),
        in_specs=[
            _row_spec(_H), _row_spec(_H), _row_spec(1), _full_spec(1, _H),
            _full_spec(_H, _H),
        ],
        out_specs=_row_spec(_H),
        out_shape=jax.ShapeDtypeStruct((_N, _H), jnp.float32),
    )(sacc, g, dis, b, wc)


def _tc_heads(sacc, g, dis, b, wd1, bd1, wd2, bd2, wi1, bi1, wi2, bi2):
    return pl.pallas_call(
        _tc3_body,
        grid=(_GRID,),
        in_specs=[
            _row_spec(_H), _row_spec(_H), _row_spec(1), _full_spec(1, _H),
            _full_spec(_H, _H // 2), _full_spec(1, _H // 2),
            _full_spec(_H // 2, 1), _full_spec(1, 1),
            _full_spec(_H, _H // 2), _full_spec(1, _H // 2),
            _full_spec(_H // 2, 1), _full_spec(1, 1),
        ],
        out_specs=[_row_spec(1), _row_spec(1)],
        out_shape=[
            jax.ShapeDtypeStruct((_N, 1), jnp.float32),
            jax.ShapeDtypeStruct((_N, 1), jnp.float32),
        ],
    )(sacc, g, dis, b, wd1, bd1, wd2, bd2, wi1, bi1, wi2, bi2)


def _to_sc(g):
    return jnp.transpose(g.reshape(_N, 4, 16), (1, 0, 2)).reshape(4 * _N, 16)


def _from_sc(out):
    o = out.reshape(4, _NPAD, 16)[:, :_N, :]
    return jnp.transpose(o, (1, 0, 2)).reshape(_N, _H)


def kernel(x, edge_index, edge_attr, W_ne, b_ne, W_ee, b_ee,
           Wc0, bc0, Wc1, bc1, Wc2, bc2,
           Wd1, bd1, Wd2, bd2, Wi1, bi1, Wi2, bi2):
    src = edge_index[0]
    dst = edge_index[1]
    pad = _EPAD - _E
    srcp = jnp.concatenate([src, jnp.zeros((pad,), jnp.int32)])
    dstp = jnp.concatenate([dst, jnp.full((pad,), _N, jnp.int32)])
    offs = (jnp.arange(4, dtype=jnp.int32) * _N)[:, None]
    srcp4 = (srcp[None, :] + offs).reshape(4, _NBLK, _BLK)
    dstp2 = dstp.reshape(_NBLK, _BLK)
    sc_degree, sc_edge_pass = _sc_kernels()

    degout = sc_degree(dstp2)
    deg = degout[:_NPAD, :1][: _N] + degout[_NPAD:, :1][: _N] + 1.0
    dis = lax.rsqrt(deg)

    g = _tc_encode(x, W_ne, b_ne.reshape(1, _H), Wc0, dis)

    for b_prev, wc_next in ((bc0, Wc1), (bc1, Wc2)):
        sacc = _from_sc(sc_edge_pass(_to_sc(g), srcp4, dstp2))
        g = _tc_layer(sacc, g, dis, b_prev.reshape(1, _H), wc_next)

    sacc = _from_sc(sc_edge_pass(_to_sc(g), srcp4, dstp2))
    demand, inv = _tc_heads(
        sacc, g, dis, bc2.reshape(1, _H),
        Wd1, bd1.reshape(1, _H // 2), Wd2, bd2.reshape(1, 1),
        Wi1, bi1.reshape(1, _H // 2), Wi2, bi2.reshape(1, 1),
    )
    return (demand, inv)


# speedup vs baseline: 11.4927x; 1.1579x over previous
import functools

import jax
import jax.numpy as jnp
from jax import lax
from jax.experimental import pallas as pl
from jax.experimental.pallas import tpu as pltpu
from jax.experimental.pallas import tpu_sc as plsc

_N = 100000
_H = 64
_NTILES = 16
_NPAD = 100352
_STRIPE = _NPAD // _NTILES
_E = 1600000

_BLK = 128
_SBB = 4
_NSB = 200
_BPT = _SBB * _NSB
_EPAD = _NTILES * _BPT * _BLK
_NBLK = _EPAD // _BLK
_NSB2 = _NSB // 2
_DROWS = 392


def _fill(buf, rows, value):
    vec = jnp.full((16,), value, jnp.float32)

    def body(i, _):
        buf[i, :] = vec
        return 0

    lax.fori_loop(0, rows, body, 0)


def _zero_accum(accum, zbuf, s):
    def body(k, _):
        pltpu.sync_copy(zbuf, accum.at[pl.ds(s * _STRIPE + k * _DROWS, _DROWS)])
        return 0

    lax.fori_loop(0, _STRIPE // _DROWS, body, 0)


def _dump_accum(accum, dbuf, out, out_row0, s):
    def body(k, _):
        r0 = s * _STRIPE + k * _DROWS
        pltpu.sync_copy(accum.at[pl.ds(r0, _DROWS)], dbuf)
        pltpu.sync_copy(dbuf, out.at[pl.ds(out_row0 + r0, _DROWS)])
        return 0

    lax.fori_loop(0, _STRIPE // _DROWS, body, 0)


def _sc_degree_body(dstp2, out, didx, obuf, dbuf, accum, sem_i, sem_s):
    c = lax.axis_index("c")
    s = lax.axis_index("s")
    _fill(obuf, _BLK, 1.0)
    _fill(dbuf, _DROWS, 0.0)
    _zero_accum(accum, dbuf, s)
    plsc.subcore_barrier()

    base = (c * _NTILES + s) * (_NSB2 * _SBB)

    def fire_idx(i, slot):
        pltpu.async_copy(dstp2.at[pl.ds(base + i * _SBB, _SBB)],
                         didx.at[slot], sem_i)

    def fire_scatters(slot):
        for b in range(_SBB):
            pltpu.async_copy(obuf, accum.at[didx.at[slot].at[b]], sem_s,
                             add=True)

    def drain_idx(slot):
        pltpu.make_async_copy(dstp2.at[pl.ds(0, _SBB)], didx.at[slot],
                              sem_i).wait()

    def drain_scatters(slot):
        for b in range(_SBB):
            pltpu.make_async_copy(obuf, accum.at[didx.at[slot].at[b]],
                                  sem_s).wait()

    fire_idx(0, 0)
    drain_idx(0)
    fire_idx(1, 1)
    fire_scatters(0)

    def body(i, _):
        slot = lax.rem(i, 2)
        prev = 1 - slot
        drain_idx(slot)

        @pl.when(i < _NSB2 - 1)
        def _():
            fire_idx(i + 1, prev)

        drain_scatters(prev)
        fire_scatters(slot)
        return 0

    lax.fori_loop(1, _NSB2, body, 0)
    drain_scatters((_NSB2 - 1) % 2)
    plsc.subcore_barrier()
    _dump_accum(accum, dbuf, out, c * _NPAD, s)


def _sc_edge_body(g4, srcp2, dstp2, out, sidx, didx, rows, dbuf, accum,
                  sem_i, sem_g, sem_s):
    c = lax.axis_index("c")
    s = lax.axis_index("s")
    base = s * _BPT

    for chunk in range(2):
        cidx = c * 2 + chunk
        off = cidx * _N
        _fill(dbuf, _DROWS, 0.0)
        _zero_accum(accum, dbuf, s)
        plsc.subcore_barrier()

        def fire_idx(i, slot):
            blk0 = base + i * _SBB
            pltpu.async_copy(srcp2.at[pl.ds(blk0, _SBB)], sidx.at[slot],
                             sem_i)
            pltpu.async_copy(dstp2.at[pl.ds(blk0, _SBB)], didx.at[slot],
                             sem_i)

        def drain_idx(slot):
            pltpu.make_async_copy(dstp2.at[pl.ds(0, _SBB)], sidx.at[slot],
                                  sem_i).wait()
            pltpu.make_async_copy(dstp2.at[pl.ds(0, _SBB)], didx.at[slot],
                                  sem_i).wait()

        def offset_idx(slot):
            for q in range(_SBB):
                for v in range(_BLK // 16):
                    sl = sidx[slot, q, pl.ds(v * 16, 16)]
                    sidx[slot, q, pl.ds(v * 16, 16)] = sl + off

        def fire_gathers(slot):
            for b in range(_SBB):
                pltpu.async_copy(g4.at[sidx.at[slot].at[b]],
                                 rows.at[slot].at[b], sem_g)

        def drain_gathers(slot):
            for b in range(_SBB):
                pltpu.make_async_copy(g4.at[sidx.at[slot].at[b]],
                                      rows.at[slot].at[b], sem_g).wait()

        def fire_scatters(slot):
            for b in range(_SBB):
                pltpu.async_copy(rows.at[slot].at[b],
                                 accum.at[didx.at[slot].at[b]], sem_s,
                                 add=True)

        def drain_scatters(slot):
            for b in range(_SBB):
                pltpu.make_async_copy(rows.at[slot].at[b],
                                      accum.at[didx.at[slot].at[b]],
                                      sem_s).wait()

        fire_idx(0, 0)
        drain_idx(0)
        offset_idx(0)
        fire_idx(1, 1)
        fire_gathers(0)
        drain_gathers(0)
        fire_scatters(0)

        def body(i, _):
            slot = lax.rem(i, 2)
            prev = 1 - slot
            drain_idx(slot)
            offset_idx(slot)

            @pl.when(i < _NSB - 1)
            def _():
                fire_idx(i + 1, prev)

            fire_gathers(slot)
            drain_scatters(prev)
            drain_gathers(slot)
            fire_scatters(slot)
            return 0

        lax.fori_loop(1, _NSB, body, 0)
        drain_scatters((_NSB - 1) % 2)
        plsc.subcore_barrier()
        _dump_accum(accum, dbuf, out, cidx * _NPAD, s)
        plsc.subcore_barrier()


@functools.cache
def _sc_kernels():
    mesh = plsc.VectorSubcoreMesh(
        core_axis_name="c", subcore_axis_name="s", num_cores=2, num_subcores=16
    )
    params = pltpu.CompilerParams(use_tc_tiling_on_sc=False)
    sc_degree = pl.kernel(
        _sc_degree_body,
        out_type=jax.ShapeDtypeStruct((2 * _NPAD, 16), jnp.float32),
        mesh=mesh,
        scratch_types=[
            pltpu.VMEM((2, _SBB, _BLK), jnp.int32),
            pltpu.VMEM((_BLK, 16), jnp.float32),
            pltpu.VMEM((_DROWS, 16), jnp.float32),
            pltpu.VMEM_SHARED((_NPAD, 16), jnp.float32),
            pltpu.SemaphoreType.DMA,
            pltpu.SemaphoreType.DMA,
        ],
        compiler_params=params,
    )
    sc_edge_pass = pl.kernel(
        _sc_edge_body,
        out_type=jax.ShapeDtypeStruct((4 * _NPAD, 16), jnp.float32),
        mesh=mesh,
        scratch_types=[
            pltpu.VMEM((2, _SBB, _BLK), jnp.int32),
            pltpu.VMEM((2, _SBB, _BLK), jnp.int32),
            pltpu.VMEM((2, _SBB, _BLK, 16), jnp.float32),
            pltpu.VMEM((_DROWS, 16), jnp.float32),
            pltpu.VMEM_SHARED((_NPAD, 16), jnp.float32),
            pltpu.SemaphoreType.DMA,
            pltpu.SemaphoreType.DMA,
            pltpu.SemaphoreType.DMA,
        ],
        compiler_params=params,
    )
    return sc_degree, sc_edge_pass


_RB = 2000
_GRID = _N // _RB


def _split4(g_ref, val):
    for cc in range(4):
        g_ref[cc] = val[:, cc * 16:(cc + 1) * 16]


def _merge4(s4, g4):
    return jnp.concatenate(
        [s4[cc] + g4[cc] for cc in range(4)], axis=1)


def _tc1_body(x_ref, wne_ref, bne_ref, wc_ref, deg_ref, g_ref, dis_ref):
    deg = deg_ref[0, :, :1] + deg_ref[1, :, :1] + 1.0
    dis = lax.rsqrt(deg)
    h = jnp.maximum(x_ref[...] @ wne_ref[...] + bne_ref[...], 0.0)
    _split4(g_ref, (h @ wc_ref[...]) * dis)
    dis_ref[...] = dis


def _tc2_body(s_ref, g_ref, dis_ref, b_ref, wc_ref, o_ref):
    dis = dis_ref[...]
    h = jnp.maximum(dis * _merge4(s_ref, g_ref) + b_ref[...], 0.0)
    _split4(o_ref, (h @ wc_ref[...]) * dis)


def _tc3_body(s_ref, g_ref, dis_ref, b_ref, wd1_ref, bd1_ref, wd2_ref,
              bd2_ref, wi1_ref, bi1_ref, wi2_ref, bi2_ref, d_ref, i_ref):
    dis = dis_ref[...]
    h = jnp.maximum(dis * _merge4(s_ref, g_ref) + b_ref[...], 0.0)
    t = jnp.maximum(h @ wd1_ref[...] + bd1_ref[...], 0.0)
    d_ref[...] = t @ wd2_ref[...] + bd2_ref[...]
    u = jnp.maximum(h @ wi1_ref[...] + bi1_ref[...], 0.0)
    i_ref[...] = u @ wi2_ref[...] + bi2_ref[...]


def _row_spec(cols):
    return pl.BlockSpec((_RB, cols), lambda i: (i, 0))


def _c4_spec():
    return pl.BlockSpec((4, _RB, 16), lambda i: (0, i, 0))


def _full_spec(r, c):
    return pl.BlockSpec((r, c), lambda i: (0, 0))


def _tc_encode(x, wne, bne, wc0, degout):
    return pl.pallas_call(
        _tc1_body,
        grid=(_GRID,),
        in_specs=[
            _row_spec(5), _full_spec(5, _H), _full_spec(1, _H),
            _full_spec(_H, _H),
            pl.BlockSpec((2, _RB, 16), lambda i: (0, i, 0)),
        ],
        out_specs=[_c4_spec(), _row_spec(1)],
        out_shape=[
            jax.ShapeDtypeStruct((4, _N, 16), jnp.float32),
            jax.ShapeDtypeStruct((_N, 1), jnp.float32),
        ],
    )(x, wne, bne, wc0, degout)


def _tc_layer(s4, g4, dis, b, wc):
    return pl.pallas_call(
        _tc2_body,
        grid=(_GRID,),
        in_specs=[
            _c4_spec(), _c4_spec(), _row_spec(1), _full_spec(1, _H),
            _full_spec(_H, _H),
        ],
        out_specs=_c4_spec(),
        out_shape=jax.ShapeDtypeStruct((4, _N, 16), jnp.float32),
    )(s4, g4, dis, b, wc)


def _tc_heads(s4, g4, dis, b, wd1, bd1, wd2, bd2, wi1, bi1, wi2, bi2):
    return pl.pallas_call(
        _tc3_body,
        grid=(_GRID,),
        in_specs=[
            _c4_spec(), _c4_spec(), _row_spec(1), _full_spec(1, _H),
            _full_spec(_H, _H // 2), _full_spec(1, _H // 2),
            _full_spec(_H // 2, 1), _full_spec(1, 1),
            _full_spec(_H, _H // 2), _full_spec(1, _H // 2),
            _full_spec(_H // 2, 1), _full_spec(1, 1),
        ],
        out_specs=[_row_spec(1), _row_spec(1)],
        out_shape=[
            jax.ShapeDtypeStruct((_N, 1), jnp.float32),
            jax.ShapeDtypeStruct((_N, 1), jnp.float32),
        ],
    )(s4, g4, dis, b, wd1, bd1, wd2, bd2, wi1, bi1, wi2, bi2)


def kernel(x, edge_index, edge_attr, W_ne, b_ne, W_ee, b_ee,
           Wc0, bc0, Wc1, bc1, Wc2, bc2,
           Wd1, bd1, Wd2, bd2, Wi1, bi1, Wi2, bi2):
    src = edge_index[0]
    dst = edge_index[1]
    pad = _EPAD - _E
    srcp2 = jnp.concatenate(
        [src, jnp.zeros((pad,), jnp.int32)]).reshape(_NBLK, _BLK)
    dstp2 = jnp.concatenate(
        [dst, jnp.full((pad,), _N, jnp.int32)]).reshape(_NBLK, _BLK)
    sc_degree, sc_edge_pass = _sc_kernels()

    degout = sc_degree(dstp2).reshape(2, _NPAD, 16)
    g4, dis = _tc_encode(x, W_ne, b_ne.reshape(1, _H), Wc0, degout)

    for b_prev, wc_next in ((bc0, Wc1), (bc1, Wc2)):
        s4 = sc_edge_pass(
            g4.reshape(4 * _N, 16), srcp2, dstp2).reshape(4, _NPAD, 16)
        g4 = _tc_layer(s4, g4, dis, b_prev.reshape(1, _H), wc_next)

    s4 = sc_edge_pass(
        g4.reshape(4 * _N, 16), srcp2, dstp2).reshape(4, _NPAD, 16)
    demand, inv = _tc_heads(
        s4, g4, dis, bc2.reshape(1, _H),
        Wd1, bd1.reshape(1, _H // 2), Wd2, bd2.reshape(1, 1),
        Wi1, bi1.reshape(1, _H // 2), Wi2, bi2.reshape(1, 1),
    )
    return (demand, inv)


# speedup vs baseline: 11.5589x; 1.0058x over previous
import functools

import jax
import jax.numpy as jnp
from jax import lax
from jax.experimental import pallas as pl
from jax.experimental.pallas import tpu as pltpu
from jax.experimental.pallas import tpu_sc as plsc

_N = 100000
_H = 64
_NTILES = 16
_NPAD = 100352
_STRIPE = _NPAD // _NTILES
_E = 1600000

_BLK = 128
_SBB = 4
_NSB = 200
_BPT = _SBB * _NSB
_EPAD = _NTILES * _BPT * _BLK
_NBLK = _EPAD // _BLK
_NSB2 = _NSB // 2
_DROWS = 392


def _fill(buf, rows, value):
    vec = jnp.full((16,), value, jnp.float32)

    def body(i, _):
        buf[i, :] = vec
        return 0

    lax.fori_loop(0, rows, body, 0)


def _zero_accum(accum, zbuf, s):
    def body(k, _):
        pltpu.sync_copy(zbuf, accum.at[pl.ds(s * _STRIPE + k * _DROWS, _DROWS)])
        return 0

    lax.fori_loop(0, _STRIPE // _DROWS, body, 0)


def _dump_accum(accum, dbuf, out, out_row0, s):
    def body(k, _):
        r0 = s * _STRIPE + k * _DROWS
        pltpu.sync_copy(accum.at[pl.ds(r0, _DROWS)], dbuf)
        pltpu.sync_copy(dbuf, out.at[pl.ds(out_row0 + r0, _DROWS)])
        return 0

    lax.fori_loop(0, _STRIPE // _DROWS, body, 0)


def _sc_degree_body(dstp2, out, didx, obuf, dbuf, accum, sem_i, sem_s):
    c = lax.axis_index("c")
    s = lax.axis_index("s")
    _fill(obuf, _BLK, 1.0)
    _fill(dbuf, _DROWS, 0.0)
    _zero_accum(accum, dbuf, s)
    plsc.subcore_barrier()

    base = (c * _NTILES + s) * (_NSB2 * _SBB)

    def fire_idx(i, slot):
        pltpu.async_copy(dstp2.at[pl.ds(base + i * _SBB, _SBB)],
                         didx.at[slot], sem_i)

    def fire_scatters(slot):
        for b in range(_SBB):
            pltpu.async_copy(obuf, accum.at[didx.at[slot].at[b]], sem_s,
                             add=True)

    def drain_idx(slot):
        pltpu.make_async_copy(dstp2.at[pl.ds(0, _SBB)], didx.at[slot],
                              sem_i).wait()

    def drain_scatters(slot):
        for b in range(_SBB):
            pltpu.make_async_copy(obuf, accum.at[didx.at[slot].at[b]],
                                  sem_s).wait()

    fire_idx(0, 0)
    drain_idx(0)
    fire_idx(1, 1)
    fire_scatters(0)

    def body(i, _):
        slot = lax.rem(i, 2)
        prev = 1 - slot
        drain_idx(slot)

        @pl.when(i < _NSB2 - 1)
        def _():
            fire_idx(i + 1, prev)

        drain_scatters(prev)
        fire_scatters(slot)
        return 0

    lax.fori_loop(1, _NSB2, body, 0)
    drain_scatters((_NSB2 - 1) % 2)
    plsc.subcore_barrier()
    _dump_accum(accum, dbuf, out.at[c], 0, s)


def _sc_edge_body(g4, srcp2, dstp2, out, sidx, didx, rows, dbuf, accum,
                  sem_i, sem_g, sem_s):
    c = lax.axis_index("c")
    s = lax.axis_index("s")
    base = s * _BPT

    for chunk in range(2):
        cidx = c * 2 + chunk
        _fill(dbuf, _DROWS, 0.0)
        _zero_accum(accum, dbuf, s)
        plsc.subcore_barrier()

        def fire_idx(i, slot):
            blk0 = base + i * _SBB
            pltpu.async_copy(srcp2.at[pl.ds(blk0, _SBB)], sidx.at[slot],
                             sem_i)
            pltpu.async_copy(dstp2.at[pl.ds(blk0, _SBB)], didx.at[slot],
                             sem_i)

        def drain_idx(slot):
            pltpu.make_async_copy(dstp2.at[pl.ds(0, _SBB)], sidx.at[slot],
                                  sem_i).wait()
            pltpu.make_async_copy(dstp2.at[pl.ds(0, _SBB)], didx.at[slot],
                                  sem_i).wait()

        def fire_gathers(slot):
            for b in range(_SBB):
                pltpu.async_copy(g4.at[cidx].at[sidx.at[slot].at[b]],
                                 rows.at[slot].at[b], sem_g)

        def drain_gathers(slot):
            for b in range(_SBB):
                pltpu.make_async_copy(g4.at[cidx].at[sidx.at[slot].at[b]],
                                      rows.at[slot].at[b], sem_g).wait()

        def fire_scatters(slot):
            for b in range(_SBB):
                pltpu.async_copy(rows.at[slot].at[b],
                                 accum.at[didx.at[slot].at[b]], sem_s,
                                 add=True)

        def drain_scatters(slot):
            for b in range(_SBB):
                pltpu.make_async_copy(rows.at[slot].at[b],
                                      accum.at[didx.at[slot].at[b]],
                                      sem_s).wait()

        fire_idx(0, 0)
        drain_idx(0)
        fire_idx(1, 1)
        fire_gathers(0)
        drain_gathers(0)
        fire_scatters(0)

        def body(i, _):
            slot = lax.rem(i, 2)
            prev = 1 - slot
            drain_idx(slot)

            @pl.when(i < _NSB - 1)
            def _():
                fire_idx(i + 1, prev)

            fire_gathers(slot)
            drain_scatters(prev)
            drain_gathers(slot)
            fire_scatters(slot)
            return 0

        lax.fori_loop(1, _NSB, body, 0)
        drain_scatters((_NSB - 1) % 2)
        plsc.subcore_barrier()
        _dump_accum(accum, dbuf, out.at[cidx], 0, s)
        plsc.subcore_barrier()


@functools.cache
def _sc_kernels():
    mesh = plsc.VectorSubcoreMesh(
        core_axis_name="c", subcore_axis_name="s", num_cores=2, num_subcores=16
    )
    params = pltpu.CompilerParams(use_tc_tiling_on_sc=False)
    sc_degree = pl.kernel(
        _sc_degree_body,
        out_type=jax.ShapeDtypeStruct((2, _NPAD, 16), jnp.float32),
        mesh=mesh,
        scratch_types=[
            pltpu.VMEM((2, _SBB, _BLK), jnp.int32),
            pltpu.VMEM((_BLK, 16), jnp.float32),
            pltpu.VMEM((_DROWS, 16), jnp.float32),
            pltpu.VMEM_SHARED((_NPAD, 16), jnp.float32),
            pltpu.SemaphoreType.DMA,
            pltpu.SemaphoreType.DMA,
        ],
        compiler_params=params,
    )
    sc_edge_pass = pl.kernel(
        _sc_edge_body,
        out_type=jax.ShapeDtypeStruct((4, _NPAD, 16), jnp.float32),
        mesh=mesh,
        scratch_types=[
            pltpu.VMEM((2, _SBB, _BLK), jnp.int32),
            pltpu.VMEM((2, _SBB, _BLK), jnp.int32),
            pltpu.VMEM((2, _SBB, _BLK, 16), jnp.float32),
            pltpu.VMEM((_DROWS, 16), jnp.float32),
            pltpu.VMEM_SHARED((_NPAD, 16), jnp.float32),
            pltpu.SemaphoreType.DMA,
            pltpu.SemaphoreType.DMA,
            pltpu.SemaphoreType.DMA,
        ],
        compiler_params=params,
    )
    return sc_degree, sc_edge_pass


_RB = 2000
_GRID = _N // _RB


def _split4(g_ref, val):
    for cc in range(4):
        g_ref[cc] = val[:, cc * 16:(cc + 1) * 16]


def _merge4(s4, g4):
    return jnp.concatenate(
        [s4[cc] + g4[cc] for cc in range(4)], axis=1)


def _tc1_body(x_ref, wne_ref, bne_ref, wc_ref, deg_ref, g_ref, dis_ref):
    deg = deg_ref[0, :, :1] + deg_ref[1, :, :1] + 1.0
    dis = lax.rsqrt(deg)
    h = jnp.maximum(x_ref[...] @ wne_ref[...] + bne_ref[...], 0.0)
    _split4(g_ref, (h @ wc_ref[...]) * dis)
    dis_ref[...] = dis


def _tc2_body(s_ref, g_ref, dis_ref, b_ref, wc_ref, o_ref):
    dis = dis_ref[...]
    h = jnp.maximum(dis * _merge4(s_ref, g_ref) + b_ref[...], 0.0)
    _split4(o_ref, (h @ wc_ref[...]) * dis)


def _tc3_body(s_ref, g_ref, dis_ref, b_ref, wd1_ref, bd1_ref, wd2_ref,
              bd2_ref, wi1_ref, bi1_ref, wi2_ref, bi2_ref, d_ref, i_ref):
    dis = dis_ref[...]
    h = jnp.maximum(dis * _merge4(s_ref, g_ref) + b_ref[...], 0.0)
    t = jnp.maximum(h @ wd1_ref[...] + bd1_ref[...], 0.0)
    d_ref[...] = t @ wd2_ref[...] + bd2_ref[...]
    u = jnp.maximum(h @ wi1_ref[...] + bi1_ref[...], 0.0)
    i_ref[...] = u @ wi2_ref[...] + bi2_ref[...]


def _row_spec(cols):
    return pl.BlockSpec((_RB, cols), lambda i: (i, 0))


def _c4_spec():
    return pl.BlockSpec((4, _RB, 16), lambda i: (0, i, 0))


def _full_spec(r, c):
    return pl.BlockSpec((r, c), lambda i: (0, 0))


def _tc_encode(x, wne, bne, wc0, degout):
    return pl.pallas_call(
        _tc1_body,
        grid=(_GRID,),
        in_specs=[
            _row_spec(5), _full_spec(5, _H), _full_spec(1, _H),
            _full_spec(_H, _H),
            pl.BlockSpec((2, _RB, 16), lambda i: (0, i, 0)),
        ],
        out_specs=[_c4_spec(), _row_spec(1)],
        out_shape=[
            jax.ShapeDtypeStruct((4, _N, 16), jnp.float32),
            jax.ShapeDtypeStruct((_N, 1), jnp.float32),
        ],
    )(x, wne, bne, wc0, degout)


def _tc_layer(s4, g4, dis, b, wc):
    return pl.pallas_call(
        _tc2_body,
        grid=(_GRID,),
        in_specs=[
            _c4_spec(), _c4_spec(), _row_spec(1), _full_spec(1, _H),
            _full_spec(_H, _H),
        ],
        out_specs=_c4_spec(),
        out_shape=jax.ShapeDtypeStruct((4, _N, 16), jnp.float32),
    )(s4, g4, dis, b, wc)


def _tc_heads(s4, g4, dis, b, wd1, bd1, wd2, bd2, wi1, bi1, wi2, bi2):
    return pl.pallas_call(
        _tc3_body,
        grid=(_GRID,),
        in_specs=[
            _c4_spec(), _c4_spec(), _row_spec(1), _full_spec(1, _H),
            _full_spec(_H, _H // 2), _full_spec(1, _H // 2),
            _full_spec(_H // 2, 1), _full_spec(1, 1),
            _full_spec(_H, _H // 2), _full_spec(1, _H // 2),
            _full_spec(_H // 2, 1), _full_spec(1, 1),
        ],
        out_specs=[_row_spec(1), _row_spec(1)],
        out_shape=[
            jax.ShapeDtypeStruct((_N, 1), jnp.float32),
            jax.ShapeDtypeStruct((_N, 1), jnp.float32),
        ],
    )(s4, g4, dis, b, wd1, bd1, wd2, bd2, wi1, bi1, wi2, bi2)


def kernel(x, edge_index, edge_attr, W_ne, b_ne, W_ee, b_ee,
           Wc0, bc0, Wc1, bc1, Wc2, bc2,
           Wd1, bd1, Wd2, bd2, Wi1, bi1, Wi2, bi2):
    src = edge_index[0]
    dst = edge_index[1]
    pad = _EPAD - _E
    srcp2 = jnp.concatenate(
        [src, jnp.zeros((pad,), jnp.int32)]).reshape(_NBLK, _BLK)
    dstp2 = jnp.concatenate(
        [dst, jnp.full((pad,), _N, jnp.int32)]).reshape(_NBLK, _BLK)
    sc_degree, sc_edge_pass = _sc_kernels()

    degout = sc_degree(dstp2)
    g4, dis = _tc_encode(x, W_ne, b_ne.reshape(1, _H), Wc0, degout)

    for b_prev, wc_next in ((bc0, Wc1), (bc1, Wc2)):
        s4 = sc_edge_pass(g4, srcp2, dstp2)
        g4 = _tc_layer(s4, g4, dis, b_prev.reshape(1, _H), wc_next)

    s4 = sc_edge_pass(g4, srcp2, dstp2)
    demand, inv = _tc_heads(
        s4, g4, dis, bc2.reshape(1, _H),
        Wd1, bd1.reshape(1, _H // 2), Wd2, bd2.reshape(1, 1),
        Wi1, bi1.reshape(1, _H // 2), Wi2, bi2.reshape(1, 1),
    )
    return (demand, inv)


# speedup vs baseline: 11.8038x; 1.0212x over previous
import functools

import jax
import jax.numpy as jnp
from jax import lax
from jax.experimental import pallas as pl
from jax.experimental.pallas import tpu as pltpu
from jax.experimental.pallas import tpu_sc as plsc

_N = 100000
_H = 64
_NTILES = 16
_NPAD = 100352
_STRIPE = _NPAD // _NTILES
_E = 1600000

_BLK = 128
_SBB = 5
_NSB = 160
_BPT = _SBB * _NSB
_EPAD = _NTILES * _BPT * _BLK
_NBLK = _EPAD // _BLK
_NSB2 = _NSB // 2
_DROWS = 196


def _fill(buf, rows, value):
    vec = jnp.full((16,), value, jnp.float32)

    def body(i, _):
        buf[i, :] = vec
        return 0

    lax.fori_loop(0, rows, body, 0)


def _zero_accum(accum, zbuf, s):
    def body(k, _):
        pltpu.sync_copy(zbuf, accum.at[pl.ds(s * _STRIPE + k * _DROWS, _DROWS)])
        return 0

    lax.fori_loop(0, _STRIPE // _DROWS, body, 0)


def _dump_accum(accum, dbuf, out, out_row0, s):
    def body(k, _):
        r0 = s * _STRIPE + k * _DROWS
        pltpu.sync_copy(accum.at[pl.ds(r0, _DROWS)], dbuf)
        pltpu.sync_copy(dbuf, out.at[pl.ds(out_row0 + r0, _DROWS)])
        return 0

    lax.fori_loop(0, _STRIPE // _DROWS, body, 0)


def _sc_degree_body(dstp2, out, didx, obuf, dbuf, accum, sem_i, sem_s):
    c = lax.axis_index("c")
    s = lax.axis_index("s")
    _fill(obuf, _BLK, 1.0)
    _fill(dbuf, _DROWS, 0.0)
    _zero_accum(accum, dbuf, s)
    plsc.subcore_barrier()

    base = (c * _NTILES + s) * (_NSB2 * _SBB)

    def fire_idx(i, slot):
        pltpu.async_copy(dstp2.at[pl.ds(base + i * _SBB, _SBB)],
                         didx.at[slot], sem_i)

    def fire_scatters(slot):
        for b in range(_SBB):
            pltpu.async_copy(obuf, accum.at[didx.at[slot].at[b]], sem_s,
                             add=True)

    def drain_idx(slot):
        pltpu.make_async_copy(dstp2.at[pl.ds(0, _SBB)], didx.at[slot],
                              sem_i).wait()

    def drain_scatters(slot):
        for b in range(_SBB):
            pltpu.make_async_copy(obuf, accum.at[didx.at[slot].at[b]],
                                  sem_s).wait()

    fire_idx(0, 0)
    drain_idx(0)
    fire_idx(1, 1)
    fire_scatters(0)

    def body(i, _):
        slot = lax.rem(i, 2)
        prev = 1 - slot
        drain_idx(slot)

        @pl.when(i < _NSB2 - 1)
        def _():
            fire_idx(i + 1, prev)

        drain_scatters(prev)
        fire_scatters(slot)
        return 0

    lax.fori_loop(1, _NSB2, body, 0)
    drain_scatters((_NSB2 - 1) % 2)
    plsc.subcore_barrier()
    _dump_accum(accum, dbuf, out.at[c], 0, s)


def _sc_edge_body(g4, srcp2, dstp2, out, sidx, didx, rows, dbuf, accum,
                  sem_i, sem_g, sem_s):
    c = lax.axis_index("c")
    s = lax.axis_index("s")
    base = s * _BPT

    for chunk in range(2):
        cidx = c * 2 + chunk
        _fill(dbuf, _DROWS, 0.0)
        _zero_accum(accum, dbuf, s)
        plsc.subcore_barrier()

        def fire_idx(i, slot):
            blk0 = base + i * _SBB
            pltpu.async_copy(srcp2.at[pl.ds(blk0, _SBB)], sidx.at[slot],
                             sem_i)
            pltpu.async_copy(dstp2.at[pl.ds(blk0, _SBB)], didx.at[slot],
                             sem_i)

        def drain_idx(slot):
            pltpu.make_async_copy(dstp2.at[pl.ds(0, _SBB)], sidx.at[slot],
                                  sem_i).wait()
            pltpu.make_async_copy(dstp2.at[pl.ds(0, _SBB)], didx.at[slot],
                                  sem_i).wait()

        def fire_gathers(slot):
            for b in range(_SBB):
                pltpu.async_copy(g4.at[cidx].at[sidx.at[slot].at[b]],
                                 rows.at[slot].at[b], sem_g)

        def drain_gathers(slot):
            for b in range(_SBB):
                pltpu.make_async_copy(g4.at[cidx].at[sidx.at[slot].at[b]],
                                      rows.at[slot].at[b], sem_g).wait()

        def fire_scatters(slot):
            for b in range(_SBB):
                pltpu.async_copy(rows.at[slot].at[b],
                                 accum.at[didx.at[slot].at[b]], sem_s,
                                 add=True)

        def drain_scatters(slot):
            for b in range(_SBB):
                pltpu.make_async_copy(rows.at[slot].at[b],
                                      accum.at[didx.at[slot].at[b]],
                                      sem_s).wait()

        fire_idx(0, 0)
        drain_idx(0)
        fire_idx(1, 1)
        fire_gathers(0)
        drain_gathers(0)
        fire_scatters(0)

        def body(i, _):
            slot = lax.rem(i, 2)
            prev = 1 - slot
            drain_idx(slot)

            @pl.when(i < _NSB - 1)
            def _():
                fire_idx(i + 1, prev)

            fire_gathers(slot)
            drain_scatters(prev)
            drain_gathers(slot)
            fire_scatters(slot)
            return 0

        lax.fori_loop(1, _NSB, body, 0)
        drain_scatters((_NSB - 1) % 2)
        plsc.subcore_barrier()
        _dump_accum(accum, dbuf, out.at[cidx], 0, s)
        plsc.subcore_barrier()


@functools.cache
def _sc_kernels():
    mesh = plsc.VectorSubcoreMesh(
        core_axis_name="c", subcore_axis_name="s", num_cores=2, num_subcores=16
    )
    params = pltpu.CompilerParams(use_tc_tiling_on_sc=False)
    sc_degree = pl.kernel(
        _sc_degree_body,
        out_type=jax.ShapeDtypeStruct((2, _NPAD, 16), jnp.float32),
        mesh=mesh,
        scratch_types=[
            pltpu.VMEM((2, _SBB, _BLK), jnp.int32),
            pltpu.VMEM((_BLK, 16), jnp.float32),
            pltpu.VMEM((_DROWS, 16), jnp.float32),
            pltpu.VMEM_SHARED((_NPAD, 16), jnp.float32),
            pltpu.SemaphoreType.DMA,
            pltpu.SemaphoreType.DMA,
        ],
        compiler_params=params,
    )
    sc_edge_pass = pl.kernel(
        _sc_edge_body,
        out_type=jax.ShapeDtypeStruct((4, _NPAD, 16), jnp.float32),
        mesh=mesh,
        scratch_types=[
            pltpu.VMEM((2, _SBB, _BLK), jnp.int32),
            pltpu.VMEM((2, _SBB, _BLK), jnp.int32),
            pltpu.VMEM((2, _SBB, _BLK, 16), jnp.float32),
            pltpu.VMEM((_DROWS, 16), jnp.float32),
            pltpu.VMEM_SHARED((_NPAD, 16), jnp.float32),
            pltpu.SemaphoreType.DMA,
            pltpu.SemaphoreType.DMA,
            pltpu.SemaphoreType.DMA,
        ],
        compiler_params=params,
    )
    return sc_degree, sc_edge_pass


_RB = 2000
_GRID = _N // _RB


def _split4(g_ref, val):
    for cc in range(4):
        g_ref[cc] = val[:, cc * 16:(cc + 1) * 16]


def _merge4(s4, g4):
    return jnp.concatenate(
        [s4[cc] + g4[cc] for cc in range(4)], axis=1)


def _tc1_body(x_ref, wne_ref, bne_ref, wc_ref, deg_ref, g_ref, dis_ref):
    deg = deg_ref[0, :, :1] + deg_ref[1, :, :1] + 1.0
    dis = lax.rsqrt(deg)
    h = jnp.maximum(x_ref[...] @ wne_ref[...] + bne_ref[...], 0.0)
    _split4(g_ref, (h @ wc_ref[...]) * dis)
    dis_ref[...] = dis


def _tc2_body(s_ref, g_ref, dis_ref, b_ref, wc_ref, o_ref):
    dis = dis_ref[...]
    h = jnp.maximum(dis * _merge4(s_ref, g_ref) + b_ref[...], 0.0)
    _split4(o_ref, (h @ wc_ref[...]) * dis)


def _tc3_body(s_ref, g_ref, dis_ref, b_ref, wd1_ref, bd1_ref, wd2_ref,
              bd2_ref, wi1_ref, bi1_ref, wi2_ref, bi2_ref, d_ref, i_ref):
    dis = dis_ref[...]
    h = jnp.maximum(dis * _merge4(s_ref, g_ref) + b_ref[...], 0.0)
    t = jnp.maximum(h @ wd1_ref[...] + bd1_ref[...], 0.0)
    d_ref[...] = t @ wd2_ref[...] + bd2_ref[...]
    u = jnp.maximum(h @ wi1_ref[...] + bi1_ref[...], 0.0)
    i_ref[...] = u @ wi2_ref[...] + bi2_ref[...]


def _row_spec(cols):
    return pl.BlockSpec((_RB, cols), lambda i: (i, 0))


def _c4_spec():
    return pl.BlockSpec((4, _RB, 16), lambda i: (0, i, 0))


def _full_spec(r, c):
    return pl.BlockSpec((r, c), lambda i: (0, 0))


def _tc_encode(x, wne, bne, wc0, degout):
    return pl.pallas_call(
        _tc1_body,
        grid=(_GRID,),
        in_specs=[
            _row_spec(5), _full_spec(5, _H), _full_spec(1, _H),
            _full_spec(_H, _H),
            pl.BlockSpec((2, _RB, 16), lambda i: (0, i, 0)),
        ],
        out_specs=[_c4_spec(), _row_spec(1)],
        out_shape=[
            jax.ShapeDtypeStruct((4, _N, 16), jnp.float32),
            jax.ShapeDtypeStruct((_N, 1), jnp.float32),
        ],
    )(x, wne, bne, wc0, degout)


def _tc_layer(s4, g4, dis, b, wc):
    return pl.pallas_call(
        _tc2_body,
        grid=(_GRID,),
        in_specs=[
            _c4_spec(), _c4_spec(), _row_spec(1), _full_spec(1, _H),
            _full_spec(_H, _H),
        ],
        out_specs=_c4_spec(),
        out_shape=jax.ShapeDtypeStruct((4, _N, 16), jnp.float32),
    )(s4, g4, dis, b, wc)


def _tc_heads(s4, g4, dis, b, wd1, bd1, wd2, bd2, wi1, bi1, wi2, bi2):
    return pl.pallas_call(
        _tc3_body,
        grid=(_GRID,),
        in_specs=[
            _c4_spec(), _c4_spec(), _row_spec(1), _full_spec(1, _H),
            _full_spec(_H, _H // 2), _full_spec(1, _H // 2),
            _full_spec(_H // 2, 1), _full_spec(1, 1),
            _full_spec(_H, _H // 2), _full_spec(1, _H // 2),
            _full_spec(_H // 2, 1), _full_spec(1, 1),
        ],
        out_specs=[_row_spec(1), _row_spec(1)],
        out_shape=[
            jax.ShapeDtypeStruct((_N, 1), jnp.float32),
            jax.ShapeDtypeStruct((_N, 1), jnp.float32),
        ],
    )(s4, g4, dis, b, wd1, bd1, wd2, bd2, wi1, bi1, wi2, bi2)


def kernel(x, edge_index, edge_attr, W_ne, b_ne, W_ee, b_ee,
           Wc0, bc0, Wc1, bc1, Wc2, bc2,
           Wd1, bd1, Wd2, bd2, Wi1, bi1, Wi2, bi2):
    src = edge_index[0]
    dst = edge_index[1]
    pad = _EPAD - _E
    srcp2 = jnp.concatenate(
        [src, jnp.zeros((pad,), jnp.int32)]).reshape(_NBLK, _BLK)
    dstp2 = jnp.concatenate(
        [dst, jnp.full((pad,), _N, jnp.int32)]).reshape(_NBLK, _BLK)
    sc_degree, sc_edge_pass = _sc_kernels()

    degout = sc_degree(dstp2)
    g4, dis = _tc_encode(x, W_ne, b_ne.reshape(1, _H), Wc0, degout)

    for b_prev, wc_next in ((bc0, Wc1), (bc1, Wc2)):
        s4 = sc_edge_pass(g4, srcp2, dstp2)
        g4 = _tc_layer(s4, g4, dis, b_prev.reshape(1, _H), wc_next)

    s4 = sc_edge_pass(g4, srcp2, dstp2)
    demand, inv = _tc_heads(
        s4, g4, dis, bc2.reshape(1, _H),
        Wd1, bd1.reshape(1, _H // 2), Wd2, bd2.reshape(1, 1),
        Wi1, bi1.reshape(1, _H // 2), Wi2, bi2.reshape(1, 1),
    )
    return (demand, inv)


# speedup vs baseline: 11.8928x; 1.0075x over previous
import functools

import jax
import jax.numpy as jnp
from jax import lax
from jax.experimental import pallas as pl
from jax.experimental.pallas import tpu as pltpu
from jax.experimental.pallas import tpu_sc as plsc

_N = 100000
_H = 64
_NTILES = 16
_NPAD = 100352
_STRIPE = _NPAD // _NTILES
_E = 1600000

_BLK = 128
_SBB = 5
_NSB = 160
_BPT = _SBB * _NSB
_EPAD = _NTILES * _BPT * _BLK
_NBLK = _EPAD // _BLK
_NSB2 = _NSB // 2
_DROWS = 196


def _fill(buf, rows, value):
    vec = jnp.full((16,), value, jnp.float32)

    def body(i, _):
        buf[i, :] = vec
        return 0

    lax.fori_loop(0, rows, body, 0)


def _zero_accum(accum, zbuf, s):
    def body(k, _):
        pltpu.sync_copy(zbuf, accum.at[pl.ds(s * _STRIPE + k * _DROWS, _DROWS)])
        return 0

    lax.fori_loop(0, _STRIPE // _DROWS, body, 0)


def _dump_accum(accum, dbuf, out, out_row0, s):
    del dbuf
    r0 = s * _STRIPE
    pltpu.sync_copy(accum.at[pl.ds(r0, _STRIPE)],
                    out.at[pl.ds(out_row0 + r0, _STRIPE)])


def _sc_degree_body(dstp2, out, didx, obuf, dbuf, accum, sem_i, sem_s):
    c = lax.axis_index("c")
    s = lax.axis_index("s")
    _fill(obuf, _BLK, 1.0)
    _fill(dbuf, _DROWS, 0.0)
    _zero_accum(accum, dbuf, s)
    plsc.subcore_barrier()

    base = (c * _NTILES + s) * (_NSB2 * _SBB)

    def fire_idx(i, slot):
        pltpu.async_copy(dstp2.at[pl.ds(base + i * _SBB, _SBB)],
                         didx.at[slot], sem_i)

    def fire_scatters(slot):
        for b in range(_SBB):
            pltpu.async_copy(obuf, accum.at[didx.at[slot].at[b]], sem_s,
                             add=True)

    def drain_idx(slot):
        pltpu.make_async_copy(dstp2.at[pl.ds(0, _SBB)], didx.at[slot],
                              sem_i).wait()

    def drain_scatters(slot):
        for b in range(_SBB):
            pltpu.make_async_copy(obuf, accum.at[didx.at[slot].at[b]],
                                  sem_s).wait()

    fire_idx(0, 0)
    drain_idx(0)
    fire_idx(1, 1)
    fire_scatters(0)

    def body(i, _):
        slot = lax.rem(i, 2)
        prev = 1 - slot
        drain_idx(slot)

        @pl.when(i < _NSB2 - 1)
        def _():
            fire_idx(i + 1, prev)

        drain_scatters(prev)
        fire_scatters(slot)
        return 0

    lax.fori_loop(1, _NSB2, body, 0)
    drain_scatters((_NSB2 - 1) % 2)
    plsc.subcore_barrier()
    _dump_accum(accum, dbuf, out.at[c], 0, s)


def _sc_edge_body(g4, srcp2, dstp2, out, sidx, didx, rows, dbuf, accum,
                  sem_i, sem_g, sem_s):
    c = lax.axis_index("c")
    s = lax.axis_index("s")
    base = s * _BPT

    for chunk in range(2):
        cidx = c * 2 + chunk
        _fill(dbuf, _DROWS, 0.0)
        _zero_accum(accum, dbuf, s)
        plsc.subcore_barrier()

        def fire_idx(i, slot):
            blk0 = base + i * _SBB
            pltpu.async_copy(srcp2.at[pl.ds(blk0, _SBB)], sidx.at[slot],
                             sem_i)
            pltpu.async_copy(dstp2.at[pl.ds(blk0, _SBB)], didx.at[slot],
                             sem_i)

        def drain_idx(slot):
            pltpu.make_async_copy(dstp2.at[pl.ds(0, _SBB)], sidx.at[slot],
                                  sem_i).wait()
            pltpu.make_async_copy(dstp2.at[pl.ds(0, _SBB)], didx.at[slot],
                                  sem_i).wait()

        def fire_gathers(slot):
            for b in range(_SBB):
                pltpu.async_copy(g4.at[cidx].at[sidx.at[slot].at[b]],
                                 rows.at[slot].at[b], sem_g)

        def drain_gathers(slot):
            for b in range(_SBB):
                pltpu.make_async_copy(g4.at[cidx].at[sidx.at[slot].at[b]],
                                      rows.at[slot].at[b], sem_g).wait()

        def fire_scatters(slot):
            for b in range(_SBB):
                pltpu.async_copy(rows.at[slot].at[b],
                                 accum.at[didx.at[slot].at[b]], sem_s,
                                 add=True)

        def drain_scatters(slot):
            for b in range(_SBB):
                pltpu.make_async_copy(rows.at[slot].at[b],
                                      accum.at[didx.at[slot].at[b]],
                                      sem_s).wait()

        fire_idx(0, 0)
        drain_idx(0)
        fire_idx(1, 1)
        fire_gathers(0)
        drain_gathers(0)
        fire_scatters(0)

        def body(i, _):
            slot = lax.rem(i, 2)
            prev = 1 - slot
            drain_idx(slot)

            @pl.when(i < _NSB - 1)
            def _():
                fire_idx(i + 1, prev)

            fire_gathers(slot)
            drain_scatters(prev)
            drain_gathers(slot)
            fire_scatters(slot)
            return 0

        lax.fori_loop(1, _NSB, body, 0)
        drain_scatters((_NSB - 1) % 2)
        plsc.subcore_barrier()
        _dump_accum(accum, dbuf, out.at[cidx], 0, s)
        plsc.subcore_barrier()


@functools.cache
def _sc_kernels():
    mesh = plsc.VectorSubcoreMesh(
        core_axis_name="c", subcore_axis_name="s", num_cores=2, num_subcores=16
    )
    params = pltpu.CompilerParams(use_tc_tiling_on_sc=False)
    sc_degree = pl.kernel(
        _sc_degree_body,
        out_type=jax.ShapeDtypeStruct((2, _NPAD, 16), jnp.float32),
        mesh=mesh,
        scratch_types=[
            pltpu.VMEM((2, _SBB, _BLK), jnp.int32),
            pltpu.VMEM((_BLK, 16), jnp.float32),
            pltpu.VMEM((_DROWS, 16), jnp.float32),
            pltpu.VMEM_SHARED((_NPAD, 16), jnp.float32),
            pltpu.SemaphoreType.DMA,
            pltpu.SemaphoreType.DMA,
        ],
        compiler_params=params,
    )
    sc_edge_pass = pl.kernel(
        _sc_edge_body,
        out_type=jax.ShapeDtypeStruct((4, _NPAD, 16), jnp.float32),
        mesh=mesh,
        scratch_types=[
            pltpu.VMEM((2, _SBB, _BLK), jnp.int32),
            pltpu.VMEM((2, _SBB, _BLK), jnp.int32),
            pltpu.VMEM((2, _SBB, _BLK, 16), jnp.float32),
            pltpu.VMEM((_DROWS, 16), jnp.float32),
            pltpu.VMEM_SHARED((_NPAD, 16), jnp.float32),
            pltpu.SemaphoreType.DMA,
            pltpu.SemaphoreType.DMA,
            pltpu.SemaphoreType.DMA,
        ],
        compiler_params=params,
    )
    return sc_degree, sc_edge_pass


_RB = 2000
_GRID = _N // _RB


def _split4(g_ref, val):
    for cc in range(4):
        g_ref[cc] = val[:, cc * 16:(cc + 1) * 16]


def _merge4(s4, g4):
    return jnp.concatenate(
        [s4[cc] + g4[cc] for cc in range(4)], axis=1)


def _tc1_body(x_ref, wne_ref, bne_ref, wc_ref, deg_ref, g_ref, dis_ref):
    deg = deg_ref[0, :, :1] + deg_ref[1, :, :1] + 1.0
    dis = lax.rsqrt(deg)
    h = jnp.maximum(x_ref[...] @ wne_ref[...] + bne_ref[...], 0.0)
    _split4(g_ref, (h @ wc_ref[...]) * dis)
    dis_ref[...] = dis


def _tc2_body(s_ref, g_ref, dis_ref, b_ref, wc_ref, o_ref):
    dis = dis_ref[...]
    h = jnp.maximum(dis * _merge4(s_ref, g_ref) + b_ref[...], 0.0)
    _split4(o_ref, (h @ wc_ref[...]) * dis)


def _tc3_body(s_ref, g_ref, dis_ref, b_ref, wd1_ref, bd1_ref, wd2_ref,
              bd2_ref, wi1_ref, bi1_ref, wi2_ref, bi2_ref, d_ref, i_ref):
    dis = dis_ref[...]
    h = jnp.maximum(dis * _merge4(s_ref, g_ref) + b_ref[...], 0.0)
    t = jnp.maximum(h @ wd1_ref[...] + bd1_ref[...], 0.0)
    d_ref[...] = t @ wd2_ref[...] + bd2_ref[...]
    u = jnp.maximum(h @ wi1_ref[...] + bi1_ref[...], 0.0)
    i_ref[...] = u @ wi2_ref[...] + bi2_ref[...]


def _row_spec(cols):
    return pl.BlockSpec((_RB, cols), lambda i: (i, 0))


def _c4_spec():
    return pl.BlockSpec((4, _RB, 16), lambda i: (0, i, 0))


def _full_spec(r, c):
    return pl.BlockSpec((r, c), lambda i: (0, 0))


def _tc_encode(x, wne, bne, wc0, degout):
    return pl.pallas_call(
        _tc1_body,
        grid=(_GRID,),
        in_specs=[
            _row_spec(5), _full_spec(5, _H), _full_spec(1, _H),
            _full_spec(_H, _H),
            pl.BlockSpec((2, _RB, 16), lambda i: (0, i, 0)),
        ],
        out_specs=[_c4_spec(), _row_spec(1)],
        out_shape=[
            jax.ShapeDtypeStruct((4, _N, 16), jnp.float32),
            jax.ShapeDtypeStruct((_N, 1), jnp.float32),
        ],
    )(x, wne, bne, wc0, degout)


def _tc_layer(s4, g4, dis, b, wc):
    return pl.pallas_call(
        _tc2_body,
        grid=(_GRID,),
        in_specs=[
            _c4_spec(), _c4_spec(), _row_spec(1), _full_spec(1, _H),
            _full_spec(_H, _H),
        ],
        out_specs=_c4_spec(),
        out_shape=jax.ShapeDtypeStruct((4, _N, 16), jnp.float32),
    )(s4, g4, dis, b, wc)


def _tc_heads(s4, g4, dis, b, wd1, bd1, wd2, bd2, wi1, bi1, wi2, bi2):
    return pl.pallas_call(
        _tc3_body,
        grid=(_GRID,),
        in_specs=[
            _c4_spec(), _c4_spec(), _row_spec(1), _full_spec(1, _H),
            _full_spec(_H, _H // 2), _full_spec(1, _H // 2),
            _full_spec(_H // 2, 1), _full_spec(1, 1),
            _full_spec(_H, _H // 2), _full_spec(1, _H // 2),
            _full_spec(_H // 2, 1), _full_spec(1, 1),
        ],
        out_specs=[_row_spec(1), _row_spec(1)],
        out_shape=[
            jax.ShapeDtypeStruct((_N, 1), jnp.float32),
            jax.ShapeDtypeStruct((_N, 1), jnp.float32),
        ],
    )(s4, g4, dis, b, wd1, bd1, wd2, bd2, wi1, bi1, wi2, bi2)


def kernel(x, edge_index, edge_attr, W_ne, b_ne, W_ee, b_ee,
           Wc0, bc0, Wc1, bc1, Wc2, bc2,
           Wd1, bd1, Wd2, bd2, Wi1, bi1, Wi2, bi2):
    src = edge_index[0]
    dst = edge_index[1]
    pad = _EPAD - _E
    srcp2 = jnp.concatenate(
        [src, jnp.zeros((pad,), jnp.int32)]).reshape(_NBLK, _BLK)
    dstp2 = jnp.concatenate(
        [dst, jnp.full((pad,), _N, jnp.int32)]).reshape(_NBLK, _BLK)
    sc_degree, sc_edge_pass = _sc_kernels()

    degout = sc_degree(dstp2)
    g4, dis = _tc_encode(x, W_ne, b_ne.reshape(1, _H), Wc0, degout)

    for b_prev, wc_next in ((bc0, Wc1), (bc1, Wc2)):
        s4 = sc_edge_pass(g4, srcp2, dstp2)
        g4 = _tc_layer(s4, g4, dis, b_prev.reshape(1, _H), wc_next)

    s4 = sc_edge_pass(g4, srcp2, dstp2)
    demand, inv = _tc_heads(
        s4, g4, dis, bc2.reshape(1, _H),
        Wd1, bd1.reshape(1, _H // 2), Wd2, bd2.reshape(1, 1),
        Wi1, bi1.reshape(1, _H // 2), Wi2, bi2.reshape(1, 1),
    )
    return (demand, inv)


# speedup vs baseline: 13.5740x; 1.1414x over previous
import functools

import jax
import jax.numpy as jnp
from jax import lax
from jax.experimental import pallas as pl
from jax.experimental.pallas import tpu as pltpu
from jax.experimental.pallas import tpu_sc as plsc

_N = 100000
_H = 64
_NTILES = 16
_NPAD = 100352
_STRIPE = _NPAD // _NTILES
_E = 1600000

_BLK = 128
_SBB = 6
_NSB = 132
_BPT = _SBB * _NSB
_EPAD = _NTILES * _BPT * _BLK
_NBLK = _EPAD // _BLK
_NSB2 = _NSB // 2
_DROWS = 98


def _fill(buf, rows, value):
    vec = jnp.full((16,), value, jnp.float32)

    def body(i, _):
        buf[i, :] = vec
        return 0

    lax.fori_loop(0, rows, body, 0)


def _zero_accum(accum, zbuf, s):
    def body(k, _):
        pltpu.sync_copy(zbuf, accum.at[pl.ds(s * _STRIPE + k * _DROWS, _DROWS)])
        return 0

    lax.fori_loop(0, _STRIPE // _DROWS, body, 0)


def _dump_accum(accum, dbuf, out, out_row0, s):
    del dbuf
    r0 = s * _STRIPE
    pltpu.sync_copy(accum.at[pl.ds(r0, _STRIPE)],
                    out.at[pl.ds(out_row0 + r0, _STRIPE)])


def _sc_degree_body(dstp2, out, didx, obuf, dbuf, accum, sem_i, sem_s):
    c = lax.axis_index("c")
    s = lax.axis_index("s")
    _fill(obuf, _BLK, 1.0)
    _fill(dbuf, _DROWS, 0.0)
    _zero_accum(accum, dbuf, s)
    plsc.subcore_barrier()

    base = (c * _NTILES + s) * (_NSB2 * _SBB)

    def fire_idx(i, slot):
        pltpu.async_copy(dstp2.at[pl.ds(base + i * _SBB, _SBB)],
                         didx.at[slot], sem_i)

    def fire_scatters(slot):
        for b in range(_SBB):
            pltpu.async_copy(obuf, accum.at[didx.at[slot].at[b]], sem_s,
                             add=True)

    def drain_idx(slot):
        pltpu.make_async_copy(dstp2.at[pl.ds(0, _SBB)], didx.at[slot],
                              sem_i).wait()

    def drain_scatters(slot):
        for b in range(_SBB):
            pltpu.make_async_copy(obuf, accum.at[didx.at[slot].at[b]],
                                  sem_s).wait()

    fire_idx(0, 0)
    drain_idx(0)
    fire_idx(1, 1)
    fire_scatters(0)

    def body(i, _):
        slot = lax.rem(i, 2)
        prev = 1 - slot
        drain_idx(slot)
        drain_scatters(prev)

        @pl.when(i < _NSB2 - 1)
        def _():
            fire_idx(i + 1, prev)

        fire_scatters(slot)
        return 0

    lax.fori_loop(1, _NSB2, body, 0)
    drain_scatters((_NSB2 - 1) % 2)
    plsc.subcore_barrier()
    _dump_accum(accum, None, out.at[c], 0, s)


def _sc_edge_body(g4, srcp2, dstp2, out, sidx, didx, rows, dbuf, accum,
                  sem_i, sem_g, sem_s):
    c = lax.axis_index("c")
    s = lax.axis_index("s")
    base = s * _BPT

    for chunk in range(2):
        cidx = c * 2 + chunk
        _fill(dbuf, _DROWS, 0.0)
        _zero_accum(accum, dbuf, s)
        plsc.subcore_barrier()

        def fire_idx(i, slot):
            blk0 = base + i * _SBB
            pltpu.async_copy(srcp2.at[pl.ds(blk0, _SBB)], sidx.at[slot],
                             sem_i)
            pltpu.async_copy(dstp2.at[pl.ds(blk0, _SBB)], didx.at[slot],
                             sem_i)

        def drain_idx(slot):
            pltpu.make_async_copy(dstp2.at[pl.ds(0, _SBB)], sidx.at[slot],
                                  sem_i).wait()
            pltpu.make_async_copy(dstp2.at[pl.ds(0, _SBB)], didx.at[slot],
                                  sem_i).wait()

        def fire_gathers(slot):
            for b in range(_SBB):
                pltpu.async_copy(g4.at[cidx].at[sidx.at[slot].at[b]],
                                 rows.at[slot].at[b], sem_g)

        def drain_gathers(slot):
            for b in range(_SBB):
                pltpu.make_async_copy(g4.at[cidx].at[sidx.at[slot].at[b]],
                                      rows.at[slot].at[b], sem_g).wait()

        def fire_scatters(slot):
            for b in range(_SBB):
                pltpu.async_copy(rows.at[slot].at[b],
                                 accum.at[didx.at[slot].at[b]], sem_s,
                                 add=True)

        def drain_scatters(slot):
            for b in range(_SBB):
                pltpu.make_async_copy(rows.at[slot].at[b],
                                      accum.at[didx.at[slot].at[b]],
                                      sem_s).wait()

        fire_idx(0, 0)
        drain_idx(0)
        fire_idx(1, 1)
        fire_gathers(0)
        drain_gathers(0)
        fire_scatters(0)

        def body(i, _):
            slot = lax.rem(i, 2)
            prev = 1 - slot
            drain_idx(slot)
            fire_gathers(slot)
            drain_scatters(prev)

            @pl.when(i < _NSB - 1)
            def _():
                fire_idx(i + 1, prev)

            drain_gathers(slot)
            fire_scatters(slot)
            return 0

        lax.fori_loop(1, _NSB, body, 0)
        drain_scatters((_NSB - 1) % 2)
        plsc.subcore_barrier()
        _dump_accum(accum, None, out.at[cidx], 0, s)
        plsc.subcore_barrier()


@functools.cache
def _sc_kernels():
    mesh = plsc.VectorSubcoreMesh(
        core_axis_name="c", subcore_axis_name="s", num_cores=2, num_subcores=16
    )
    params = pltpu.CompilerParams(use_tc_tiling_on_sc=False)
    sc_degree = pl.kernel(
        _sc_degree_body,
        out_type=jax.ShapeDtypeStruct((2, _NPAD, 16), jnp.float32),
        mesh=mesh,
        scratch_types=[
            pltpu.VMEM((2, _SBB, _BLK), jnp.int32),
            pltpu.VMEM((_BLK, 16), jnp.float32),
            pltpu.VMEM((_DROWS, 16), jnp.float32),
            pltpu.VMEM_SHARED((_NPAD, 16), jnp.float32),
            pltpu.SemaphoreType.DMA,
            pltpu.SemaphoreType.DMA,
        ],
        compiler_params=params,
    )
    sc_edge_pass = pl.kernel(
        _sc_edge_body,
        out_type=jax.ShapeDtypeStruct((4, _NPAD, 16), jnp.float32),
        mesh=mesh,
        scratch_types=[
            pltpu.VMEM((2, _SBB, _BLK), jnp.int32),
            pltpu.VMEM((2, _SBB, _BLK), jnp.int32),
            pltpu.VMEM((2, _SBB, _BLK, 16), jnp.float32),
            pltpu.VMEM((_DROWS, 16), jnp.float32),
            pltpu.VMEM_SHARED((_NPAD, 16), jnp.float32),
            pltpu.SemaphoreType.DMA,
            pltpu.SemaphoreType.DMA,
            pltpu.SemaphoreType.DMA,
        ],
        compiler_params=params,
    )
    return sc_degree, sc_edge_pass


_RB = 2000
_GRID = _N // _RB


def _split4(g_ref, val):
    for cc in range(4):
        g_ref[cc] = val[:, cc * 16:(cc + 1) * 16]


def _merge4(s4, g4):
    return jnp.concatenate(
        [s4[cc] + g4[cc] for cc in range(4)], axis=1)


def _tc1_body(x_ref, wne_ref, bne_ref, wc_ref, deg_ref, g_ref, dis_ref):
    deg = deg_ref[0, :, :1] + deg_ref[1, :, :1] + 1.0
    dis = lax.rsqrt(deg)
    h = jnp.maximum(x_ref[...] @ wne_ref[...] + bne_ref[...], 0.0)
    _split4(g_ref, (h @ wc_ref[...]) * dis)
    dis_ref[...] = dis


def _tc2_body(s_ref, g_ref, dis_ref, b_ref, wc_ref, o_ref):
    dis = dis_ref[...]
    h = jnp.maximum(dis * _merge4(s_ref, g_ref) + b_ref[...], 0.0)
    _split4(o_ref, (h @ wc_ref[...]) * dis)


def _tc3_body(s_ref, g_ref, dis_ref, b_ref, wd1_ref, bd1_ref, wd2_ref,
              bd2_ref, wi1_ref, bi1_ref, wi2_ref, bi2_ref, d_ref, i_ref):
    dis = dis_ref[...]
    h = jnp.maximum(dis * _merge4(s_ref, g_ref) + b_ref[...], 0.0)
    t = jnp.maximum(h @ wd1_ref[...] + bd1_ref[...], 0.0)
    d_ref[...] = t @ wd2_ref[...] + bd2_ref[...]
    u = jnp.maximum(h @ wi1_ref[...] + bi1_ref[...], 0.0)
    i_ref[...] = u @ wi2_ref[...] + bi2_ref[...]


def _row_spec(cols):
    return pl.BlockSpec((_RB, cols), lambda i: (i, 0))


def _c4_spec():
    return pl.BlockSpec((4, _RB, 16), lambda i: (0, i, 0))


def _full_spec(r, c):
    return pl.BlockSpec((r, c), lambda i: (0, 0))


def _tc_encode(x, wne, bne, wc0, degout):
    return pl.pallas_call(
        _tc1_body,
        grid=(_GRID,),
        in_specs=[
            _row_spec(5), _full_spec(5, _H), _full_spec(1, _H),
            _full_spec(_H, _H),
            pl.BlockSpec((2, _RB, 16), lambda i: (0, i, 0)),
        ],
        out_specs=[_c4_spec(), _row_spec(1)],
        out_shape=[
            jax.ShapeDtypeStruct((4, _N, 16), jnp.float32),
            jax.ShapeDtypeStruct((_N, 1), jnp.float32),
        ],
    )(x, wne, bne, wc0, degout)


def _tc_layer(s4, g4, dis, b, wc):
    return pl.pallas_call(
        _tc2_body,
        grid=(_GRID,),
        in_specs=[
            _c4_spec(), _c4_spec(), _row_spec(1), _full_spec(1, _H),
            _full_spec(_H, _H),
        ],
        out_specs=_c4_spec(),
        out_shape=jax.ShapeDtypeStruct((4, _N, 16), jnp.float32),
    )(s4, g4, dis, b, wc)


def _tc_heads(s4, g4, dis, b, wd1, bd1, wd2, bd2, wi1, bi1, wi2, bi2):
    return pl.pallas_call(
        _tc3_body,
        grid=(_GRID,),
        in_specs=[
            _c4_spec(), _c4_spec(), _row_spec(1), _full_spec(1, _H),
            _full_spec(_H, _H // 2), _full_spec(1, _H // 2),
            _full_spec(_H // 2, 1), _full_spec(1, 1),
            _full_spec(_H, _H // 2), _full_spec(1, _H // 2),
            _full_spec(_H // 2, 1), _full_spec(1, 1),
        ],
        out_specs=[_row_spec(1), _row_spec(1)],
        out_shape=[
            jax.ShapeDtypeStruct((_N, 1), jnp.float32),
            jax.ShapeDtypeStruct((_N, 1), jnp.float32),
        ],
    )(s4, g4, dis, b, wd1, bd1, wd2, bd2, wi1, bi1, wi2, bi2)


def kernel(x, edge_index, edge_attr, W_ne, b_ne, W_ee, b_ee,
           Wc0, bc0, Wc1, bc1, Wc2, bc2,
           Wd1, bd1, Wd2, bd2, Wi1, bi1, Wi2, bi2):
    src = edge_index[0]
    dst = edge_index[1]
    pad = _EPAD - _E
    srcp2 = jnp.concatenate(
        [src, jnp.zeros((pad,), jnp.int32)]).reshape(_NBLK, _BLK)
    dstp2 = jnp.concatenate(
        [dst, jnp.full((pad,), _N, jnp.int32)]).reshape(_NBLK, _BLK)
    sc_degree, sc_edge_pass = _sc_kernels()

    degout = sc_degree(dstp2)
    g4, dis = _tc_encode(x, W_ne, b_ne.reshape(1, _H), Wc0, degout)

    for b_prev, wc_next in ((bc0, Wc1), (bc1, Wc2)):
        s4 = sc_edge_pass(g4, srcp2, dstp2)
        g4 = _tc_layer(s4, g4, dis, b_prev.reshape(1, _H), wc_next)

    s4 = sc_edge_pass(g4, srcp2, dstp2)
    demand, inv = _tc_heads(
        s4, g4, dis, bc2.reshape(1, _H),
        Wd1, bd1.reshape(1, _H // 2), Wd2, bd2.reshape(1, 1),
        Wi1, bi1.reshape(1, _H // 2), Wi2, bi2.reshape(1, 1),
    )
    return (demand, inv)
